# Initial kernel scaffold; baseline (speedup 1.0000x reference)
import jax
import jax.numpy as jnp
from jax.experimental import pallas as pl


def kernel(x, edge_index, W, b):
    raise NotImplementedError("write your pallas kernel here")


# speedup vs baseline: 4.5621x; 4.5621x over previous
import functools

import jax
import jax.numpy as jnp
from jax import lax
from jax.experimental import pallas as pl
from jax.experimental.pallas import tpu as pltpu
from jax.experimental.pallas import tpu_sc as plsc

D_IN = 128
D_AUG = 144
CHUNK = 128
NC = 2
NS = 16
NW = NC * NS


def _sc_aggregate(e_pad, chunks_per_worker, n_pad):
    mesh = plsc.VectorSubcoreMesh(core_axis_name="c", subcore_axis_name="s")
    rows_per_tile = n_pad // NS

    @functools.partial(
        pl.kernel,
        mesh=mesh,
        compiler_params=pltpu.CompilerParams(use_tc_tiling_on_sc=False),
        out_type=jax.ShapeDtypeStruct((NC, n_pad, D_AUG), jnp.float32),
        scratch_types=[
            pltpu.VMEM((CHUNK,), jnp.int32),
            pltpu.VMEM((CHUNK,), jnp.int32),
            pltpu.VMEM((CHUNK, D_AUG), jnp.float32),
            pltpu.VMEM_SHARED((n_pad, D_AUG), jnp.float32),
            pltpu.SemaphoreType.DMA,
        ],
    )
    def sc_agg(xa_hbm, src_hbm, dst_hbm, out_hbm, src_v, dst_v, rows_v,
               acc_sh, sem):
        c = lax.axis_index("c")
        s = lax.axis_index("s")
        wid = s * NC + c

        def zero_row(i, carry):
            for j in range(D_AUG // 16):
                rows_v[i, pl.ds(j * 16, 16)] = jnp.zeros((16,), jnp.float32)
            return carry

        lax.fori_loop(0, CHUNK, zero_row, 0)
        for k in range(rows_per_tile // CHUNK):
            pltpu.sync_copy(
                rows_v, acc_sh.at[pl.ds(s * rows_per_tile + k * CHUNK, CHUNK)])
        plsc.subcore_barrier()

        def body(i, carry):
            base = (wid * chunks_per_worker + i) * CHUNK
            pltpu.sync_copy(src_hbm.at[pl.ds(base, CHUNK)], src_v)
            pltpu.sync_copy(dst_hbm.at[pl.ds(base, CHUNK)], dst_v)
            pltpu.async_copy(xa_hbm.at[src_v], rows_v, sem).wait()
            pltpu.sync_copy(rows_v, acc_sh.at[dst_v], add=True)
            return carry

        lax.fori_loop(0, chunks_per_worker, body, 0)
        plsc.subcore_barrier()

        for k in range(rows_per_tile // CHUNK):
            r0 = s * rows_per_tile + k * CHUNK
            pltpu.sync_copy(acc_sh.at[pl.ds(r0, CHUNK)], rows_v)
            pltpu.sync_copy(rows_v, out_hbm.at[c, pl.ds(r0, CHUNK)])

    return sc_agg


def _tc_combine(x, agg2, W, b, n_nodes):
    blk = 1000

    def body(x_ref, a_ref, w_ref, b_ref, o_ref):
        ssum = a_ref[0] + a_ref[1]
        cnt = ssum[:, D_IN:D_IN + 1]
        agg = ssum[:, :D_IN] / (cnt + 1e-8)
        h = x_ref[...] + agg
        o_ref[...] = lax.dot_general(
            h, w_ref[...], (((1,), (1,)), ((), ())),
            preferred_element_type=jnp.float32) + b_ref[...]

    n_pad = agg2.shape[1]
    return pl.pallas_call(
        body,
        grid=(n_nodes // blk,),
        in_specs=[
            pl.BlockSpec((blk, D_IN), lambda i: (i, 0)),
            pl.BlockSpec((NC, blk, D_AUG), lambda i: (0, i, 0)),
            pl.BlockSpec((D_IN, D_IN), lambda i: (0, 0)),
            pl.BlockSpec((1, D_IN), lambda i: (0, 0)),
        ],
        out_specs=pl.BlockSpec((blk, D_IN), lambda i: (i, 0)),
        out_shape=jax.ShapeDtypeStruct((n_nodes, D_IN), jnp.float32),
    )(x, agg2, W, b.reshape(1, D_IN))


def kernel(x, edge_index, W, b):
    n = x.shape[0]
    e = edge_index.shape[1]
    grain = NW * CHUNK
    e_pad = ((e + grain - 1) // grain) * grain
    chunks_per_worker = e_pad // grain
    n_pad = ((n + 1 + NS * CHUNK - 1) // (NS * CHUNK)) * (NS * CHUNK)

    src = edge_index[0].astype(jnp.int32)
    dst = edge_index[1].astype(jnp.int32)
    pad = e_pad - e
    if pad:
        src = jnp.concatenate([src, jnp.zeros((pad,), jnp.int32)])
        dst = jnp.concatenate([dst, jnp.full((pad,), n, jnp.int32)])
    xa = jnp.concatenate(
        [x, jnp.ones((n, D_AUG - D_IN), x.dtype)], axis=1)

    agg2 = _sc_aggregate(e_pad, chunks_per_worker, n_pad)(xa, src, dst)
    return _tc_combine(x, agg2, W, b, n)
